# Initial kernel scaffold; baseline (speedup 1.0000x reference)
import jax
import jax.numpy as jnp
from jax.experimental import pallas as pl


def kernel(x):
    raise NotImplementedError("write your pallas kernel here")


# speedup vs baseline: 1.6745x; 1.6745x over previous
import jax
import jax.numpy as jnp
from jax.experimental import pallas as pl
from jax.experimental.pallas import tpu as pltpu

_NB, _NA, _ATTR = 16, 3, 85
_NH = _NW = 76
_NP = _NH * _NW
_NC = _NA * _ATTR
_STRIDE = 8.0
_AW = (116.0, 156.0, 373.0)
_AH = (90.0, 198.0, 326.0)

_TP = 512
_NTILES = (_NP + _TP - 1) // _TP


def _body(x_ref, o_ref):
    j = pl.program_id(1)
    v = x_ref[0]
    t = v.T
    c = jax.lax.broadcasted_iota(jnp.int32, (_TP, _NC), 1)
    r = c % _ATTR
    a = c // _ATTR
    is2 = r == 2
    is3 = r == 3
    isexp = is2 | is3
    e = jnp.exp(jnp.where(isexp, t, -t))
    sig = 1.0 / (1.0 + e)
    base = jnp.where(isexp, e, sig)
    aw = jnp.where(a == 0, _AW[0], jnp.where(a == 1, _AW[1], _AW[2]))
    ah = jnp.where(a == 0, _AH[0], jnp.where(a == 1, _AH[1], _AH[2]))
    mul = jnp.where(r == 0, _STRIDE,
          jnp.where(r == 1, _STRIDE,
          jnp.where(is2, aw,
          jnp.where(is3, ah, 1.0))))
    p = j * _TP + jax.lax.broadcasted_iota(jnp.int32, (_TP, _NC), 0)
    w = (p % _NW).astype(jnp.float32)
    h = (p // _NW).astype(jnp.float32)
    add = jnp.where(r == 0, _STRIDE * w,
          jnp.where(r == 1, _STRIDE * h, 0.0))
    o_ref[0] = base * mul + add


def kernel(x):
    xr = x.reshape(_NB, _NC, _NP)
    out = pl.pallas_call(
        _body,
        grid=(_NB, _NTILES),
        in_specs=[pl.BlockSpec((1, _NC, _TP), lambda b, j: (b, 0, j))],
        out_specs=pl.BlockSpec((1, _TP, _NC), lambda b, j: (b, j, 0)),
        out_shape=jax.ShapeDtypeStruct((_NB, _NP, _NC), jnp.float32),
    )(xr)
    return out.reshape(_NB, _NA * _NP, _ATTR)


# speedup vs baseline: 1.9424x; 1.1600x over previous
import jax
import jax.numpy as jnp
from jax.experimental import pallas as pl
from jax.experimental.pallas import tpu as pltpu

_NB, _NA, _ATTR = 16, 3, 85
_NH = _NW = 76
_NP = _NH * _NW
_NC = _NA * _ATTR
_STRIDE = 8.0
_AW = (116.0, 156.0, 373.0)
_AH = (90.0, 198.0, 326.0)

_TP = 2048
_NTILES = (_NP + _TP - 1) // _TP


def _body(x_ref, o_ref):
    j = pl.program_id(1)
    v = x_ref[0]
    t = v.T
    c = jax.lax.broadcasted_iota(jnp.int32, (1, _NC), 1)
    r = c % _ATTR
    a = c // _ATTR
    isexp = (r == 2) | (r == 3)
    e = jnp.exp(jnp.where(isexp, t, -t))
    base = jnp.where(isexp, e, 1.0 / (1.0 + e))
    aw = jnp.where(a == 0, _AW[0], jnp.where(a == 1, _AW[1], _AW[2]))
    ah = jnp.where(a == 0, _AH[0], jnp.where(a == 1, _AH[1], _AH[2]))
    mul = jnp.where(r < 2, _STRIDE,
          jnp.where(r == 2, aw,
          jnp.where(r == 3, ah, 1.0))).astype(jnp.float32)
    p = j * _TP + jax.lax.broadcasted_iota(jnp.int32, (_TP, 1), 0)
    w = (p % _NW).astype(jnp.float32)
    h = (p // _NW).astype(jnp.float32)
    m0 = (r == 0).astype(jnp.float32)
    m1 = (r == 1).astype(jnp.float32)
    add = m0 * (_STRIDE * w) + m1 * (_STRIDE * h)
    o_ref[0] = base * mul + add


def kernel(x):
    xr = x.reshape(_NB, _NC, _NP)
    out = pl.pallas_call(
        _body,
        grid=(_NB, _NTILES),
        in_specs=[pl.BlockSpec((1, _NC, _TP), lambda b, j: (b, 0, j))],
        out_specs=pl.BlockSpec((1, _TP, _NC), lambda b, j: (b, j, 0)),
        out_shape=jax.ShapeDtypeStruct((_NB, _NP, _NC), jnp.float32),
    )(xr)
    return out.reshape(_NB, _NA * _NP, _ATTR)
